# baseline (speedup 1.0000x reference)
import functools

import jax
import jax.numpy as jnp
from jax import lax
from jax.experimental import pallas as pl
from jax.experimental.pallas import tpu as pltpu
from jax.experimental.pallas import tpu_sc as plsc

_BATCH = 4096
_HIST = 200
_D = 64
_NW = 32
_B = _BATCH * _HIST
_BPW = _B // _NW
_C = 640
_H = _C // 2
_NCHUNK = _BPW // _C
_OPW = _BPW * _D // 128
_OROWS = _B * _D // 128


def _make_sc_gather():
    mesh = plsc.VectorSubcoreMesh(core_axis_name="c", subcore_axis_name="s")

    @functools.partial(
        pl.kernel,
        mesh=mesh,
        compiler_params=pltpu.CompilerParams(use_tc_tiling_on_sc=False),
        out_type=jax.ShapeDtypeStruct((_OROWS, 128), jnp.float32),
        scratch_types=[
            pltpu.VMEM((_C,), jnp.int32),
            pltpu.VMEM((_C,), jnp.int32),
            pltpu.VMEM((_H, _D), jnp.float32),
            pltpu.VMEM((_H, _D), jnp.float32),
            pltpu.VMEM((_H, _D), jnp.float32),
            pltpu.VMEM((_H, _D), jnp.float32),
            pltpu.SemaphoreType.DMA,
            pltpu.SemaphoreType.DMA,
            pltpu.SemaphoreType.DMA,
            pltpu.SemaphoreType.DMA,
        ],
    )
    def k(idx_hbm, table_hbm, out_hbm, idx0, idx1,
          rowsl0, rowsl1, rowsr0, rowsr1, gs0, gs1, ws0, ws1):
        wid = lax.axis_index("s") * 2 + lax.axis_index("c")
        obase = wid * _OPW
        idxb = (idx0, idx1)
        rowslb = (rowsl0, rowsl1)
        rowsrb = (rowsr0, rowsr1)
        gs = (gs0, gs1)
        ws = (ws0, ws1)

        def issue_gather(g, b):
            pltpu.sync_copy(idx_hbm.at[wid, g], idxb[b])
            pltpu.async_copy(table_hbm.at[idxb[b].at[pl.ds(0, _H)]],
                             rowslb[b], gs[b])
            pltpu.async_copy(table_hbm.at[idxb[b].at[pl.ds(_H, _H)]],
                             rowsrb[b], gs[b])

        def wait_gather(b):
            pltpu.make_async_copy(table_hbm.at[idxb[b].at[pl.ds(0, _H)]],
                                  rowslb[b], gs[b]).wait()
            pltpu.make_async_copy(table_hbm.at[idxb[b].at[pl.ds(_H, _H)]],
                                  rowsrb[b], gs[b]).wait()

        def issue_write(g, b):
            o = obase + g * _H
            pltpu.async_copy(rowslb[b],
                             out_hbm.at[pl.ds(o, _H), pl.ds(0, _D)], ws[b])
            pltpu.async_copy(rowsrb[b],
                             out_hbm.at[pl.ds(o, _H), pl.ds(_D, _D)], ws[b])

        def wait_write(g, b):
            o = obase + g * _H
            pltpu.make_async_copy(rowslb[b],
                                  out_hbm.at[pl.ds(o, _H), pl.ds(0, _D)],
                                  ws[b]).wait()
            pltpu.make_async_copy(rowsrb[b],
                                  out_hbm.at[pl.ds(o, _H), pl.ds(_D, _D)],
                                  ws[b]).wait()

        issue_gather(0, 0)

        def body(i, carry):
            for b in range(2):
                g = i * 2 + b
                nb = (b + 1) % 2
                wait_gather(b)
                issue_write(g, b)

                @pl.when(g >= 1)
                def _():
                    wait_write(g - 1, nb)

                @pl.when(g + 1 < _NCHUNK)
                def _():
                    issue_gather(g + 1, nb)
            return carry

        lax.fori_loop(0, _NCHUNK // 2, body, 0)
        wait_write(_NCHUNK - 1, (_NCHUNK - 1) % 2)

    return k


_sc_gather = _make_sc_gather()


def kernel(timestamps, table):
    idx = (timestamps.reshape(_NW, _NCHUNK, _H, 2)
           .transpose(0, 1, 3, 2)
           .reshape(_NW, _NCHUNK, _C))
    out = _sc_gather(idx, table)
    return out.reshape(_BATCH, _HIST, _D)


# speedup vs baseline: 1.2527x; 1.2527x over previous
import functools

import jax
import jax.numpy as jnp
from jax import lax
from jax.experimental import pallas as pl
from jax.experimental.pallas import tpu as pltpu
from jax.experimental.pallas import tpu_sc as plsc

_BATCH = 4096
_HIST = 200
_D = 64
_NW = 32
_B = _BATCH * _HIST
_BPW = _B // _NW
_C = 512
_NCHUNK = _BPW // _C


def _make_sc_gather():
    mesh = plsc.VectorSubcoreMesh(core_axis_name="c", subcore_axis_name="s")

    @functools.partial(
        pl.kernel,
        mesh=mesh,
        compiler_params=pltpu.CompilerParams(use_tc_tiling_on_sc=False),
        out_type=jax.ShapeDtypeStruct((_B, _D), jnp.float32),
        scratch_types=[
            pltpu.VMEM((_C,), jnp.int32),
            pltpu.VMEM((_C,), jnp.int32),
            pltpu.VMEM((_C, _D), jnp.float32),
            pltpu.VMEM((_C, _D), jnp.float32),
            pltpu.SemaphoreType.DMA,
            pltpu.SemaphoreType.DMA,
            pltpu.SemaphoreType.DMA,
            pltpu.SemaphoreType.DMA,
        ],
    )
    def k(idx_hbm, table_hbm, out_hbm, idx0, idx1, rows0, rows1,
          gs0, gs1, ws0, ws1):
        wid = lax.axis_index("s") * 2 + lax.axis_index("c")
        base = wid * _BPW
        idxb = (idx0, idx1)
        rowsb = (rows0, rows1)
        gs = (gs0, gs1)
        ws = (ws0, ws1)

        def issue_gather(g, b):
            pltpu.sync_copy(idx_hbm.at[wid, g], idxb[b])
            pltpu.async_copy(table_hbm.at[idxb[b]], rowsb[b], gs[b])

        def wait_gather(b):
            pltpu.make_async_copy(table_hbm.at[idxb[b]], rowsb[b], gs[b]).wait()

        def issue_write(g, b):
            pltpu.async_copy(rowsb[b], out_hbm.at[pl.ds(base + g * _C, _C)],
                             ws[b])

        def wait_write(g, b):
            pltpu.make_async_copy(rowsb[b],
                                  out_hbm.at[pl.ds(base + g * _C, _C)],
                                  ws[b]).wait()

        issue_gather(0, 0)

        def body(i, carry):
            for b in range(2):
                g = i * 2 + b
                nb = (b + 1) % 2
                wait_gather(b)
                issue_write(g, b)

                @pl.when(g >= 1)
                def _():
                    wait_write(g - 1, nb)

                @pl.when(g + 1 < _NCHUNK)
                def _():
                    issue_gather(g + 1, nb)
            return carry

        lax.fori_loop(0, _NCHUNK // 2, body, 0)
        wait_write(_NCHUNK - 1, (_NCHUNK - 1) % 2)

    return k


_sc_gather = _make_sc_gather()


def kernel(timestamps, table):
    idx = timestamps.reshape(_NW, _NCHUNK, _C)
    out = _sc_gather(idx, table)
    return out.reshape(_BATCH, _HIST, _D)


# speedup vs baseline: 1.2679x; 1.0121x over previous
import functools

import jax
import jax.numpy as jnp
from jax import lax
from jax.experimental import pallas as pl
from jax.experimental.pallas import tpu as pltpu
from jax.experimental.pallas import tpu_sc as plsc

_BATCH = 4096
_HIST = 200
_D = 64
_NW = 32
_B = _BATCH * _HIST
_BPW = _B // _NW
_C = 640
_NCHUNK = _BPW // _C


def _make_sc_gather():
    mesh = plsc.VectorSubcoreMesh(core_axis_name="c", subcore_axis_name="s")

    @functools.partial(
        pl.kernel,
        mesh=mesh,
        compiler_params=pltpu.CompilerParams(use_tc_tiling_on_sc=False),
        out_type=jax.ShapeDtypeStruct((_B, _D), jnp.float32),
        scratch_types=[
            pltpu.VMEM((_C,), jnp.int32),
            pltpu.VMEM((_C,), jnp.int32),
            pltpu.VMEM((_C, _D), jnp.float32),
            pltpu.VMEM((_C, _D), jnp.float32),
            pltpu.SemaphoreType.DMA,
            pltpu.SemaphoreType.DMA,
            pltpu.SemaphoreType.DMA,
            pltpu.SemaphoreType.DMA,
        ],
    )
    def k(idx_hbm, table_hbm, out_hbm, idx0, idx1, rows0, rows1,
          gs0, gs1, ws0, ws1):
        wid = lax.axis_index("s") * 2 + lax.axis_index("c")
        base = wid * _BPW
        idxb = (idx0, idx1)
        rowsb = (rows0, rows1)
        gs = (gs0, gs1)
        ws = (ws0, ws1)

        def issue_gather(g, b):
            pltpu.sync_copy(idx_hbm.at[wid, g], idxb[b])
            pltpu.async_copy(table_hbm.at[idxb[b]], rowsb[b], gs[b])

        def wait_gather(b):
            pltpu.make_async_copy(table_hbm.at[idxb[b]], rowsb[b], gs[b]).wait()

        def issue_write(g, b):
            pltpu.async_copy(rowsb[b], out_hbm.at[pl.ds(base + g * _C, _C)],
                             ws[b])

        def wait_write(g, b):
            pltpu.make_async_copy(rowsb[b],
                                  out_hbm.at[pl.ds(base + g * _C, _C)],
                                  ws[b]).wait()

        issue_gather(0, 0)

        def body(i, carry):
            for b in range(2):
                g = i * 2 + b
                nb = (b + 1) % 2
                wait_gather(b)
                issue_write(g, b)

                @pl.when(g >= 1)
                def _():
                    wait_write(g - 1, nb)

                @pl.when(g + 1 < _NCHUNK)
                def _():
                    issue_gather(g + 1, nb)
            return carry

        lax.fori_loop(0, _NCHUNK // 2, body, 0)
        wait_write(_NCHUNK - 1, (_NCHUNK - 1) % 2)

    return k


_sc_gather = _make_sc_gather()


def kernel(timestamps, table):
    idx = timestamps.reshape(_NW, _NCHUNK, _C)
    out = _sc_gather(idx, table)
    return out.reshape(_BATCH, _HIST, _D)


# speedup vs baseline: 1.2700x; 1.0016x over previous
import functools

import jax
import jax.numpy as jnp
from jax import lax
from jax.experimental import pallas as pl
from jax.experimental.pallas import tpu as pltpu
from jax.experimental.pallas import tpu_sc as plsc

_BATCH = 4096
_HIST = 200
_D = 64
_NW = 32
_B = _BATCH * _HIST
_BPW = _B // _NW
_C = 800
_NCHUNK = _BPW // _C


def _make_sc_gather():
    mesh = plsc.VectorSubcoreMesh(core_axis_name="c", subcore_axis_name="s")

    @functools.partial(
        pl.kernel,
        mesh=mesh,
        compiler_params=pltpu.CompilerParams(use_tc_tiling_on_sc=False),
        out_type=jax.ShapeDtypeStruct((_B, _D), jnp.float32),
        scratch_types=[
            pltpu.VMEM((_C,), jnp.int32),
            pltpu.VMEM((_C,), jnp.int32),
            pltpu.VMEM((_C, _D), jnp.float32),
            pltpu.VMEM((_C, _D), jnp.float32),
            pltpu.SemaphoreType.DMA,
            pltpu.SemaphoreType.DMA,
            pltpu.SemaphoreType.DMA,
            pltpu.SemaphoreType.DMA,
        ],
    )
    def k(idx_hbm, table_hbm, out_hbm, idx0, idx1, rows0, rows1,
          gs0, gs1, ws0, ws1):
        wid = lax.axis_index("s") * 2 + lax.axis_index("c")
        base = wid * _BPW
        idxb = (idx0, idx1)
        rowsb = (rows0, rows1)
        gs = (gs0, gs1)
        ws = (ws0, ws1)

        def issue_gather(g, b):
            pltpu.sync_copy(idx_hbm.at[wid, g], idxb[b])
            pltpu.async_copy(table_hbm.at[idxb[b]], rowsb[b], gs[b])

        def wait_gather(b):
            pltpu.make_async_copy(table_hbm.at[idxb[b]], rowsb[b], gs[b]).wait()

        def issue_write(g, b):
            pltpu.async_copy(rowsb[b], out_hbm.at[pl.ds(base + g * _C, _C)],
                             ws[b])

        def wait_write(g, b):
            pltpu.make_async_copy(rowsb[b],
                                  out_hbm.at[pl.ds(base + g * _C, _C)],
                                  ws[b]).wait()

        issue_gather(0, 0)

        def body(i, carry):
            for b in range(2):
                g = i * 2 + b
                nb = (b + 1) % 2
                wait_gather(b)
                issue_write(g, b)

                @pl.when(g >= 1)
                def _():
                    wait_write(g - 1, nb)

                @pl.when(g + 1 < _NCHUNK)
                def _():
                    issue_gather(g + 1, nb)
            return carry

        lax.fori_loop(0, _NCHUNK // 2, body, 0)
        wait_write(_NCHUNK - 1, (_NCHUNK - 1) % 2)

    return k


_sc_gather = _make_sc_gather()


def kernel(timestamps, table):
    idx = timestamps.reshape(_NW, _NCHUNK, _C)
    out = _sc_gather(idx, table)
    return out.reshape(_BATCH, _HIST, _D)
